# Initial kernel scaffold; baseline (speedup 1.0000x reference)
import jax
import jax.numpy as jnp
from jax.experimental import pallas as pl


def kernel(x, edge_index, edge_attr, batch, vn_emb, vn_W1, vn_b1, vn_g, vn_bt, vn_W2, vn_b2, agg_eW, agg_eb, agg_W1, agg_b1, agg_W2, agg_b2, lin_W0, lin_b0, lin_g0, lin_bt0, lin_W1, lin_b1, lin_g1, lin_bt1, emb_W, emb_b, out_W, out_b):
    raise NotImplementedError("write your pallas kernel here")


# speedup vs baseline: 3.2736x; 3.2736x over previous
import functools

import jax
import jax.numpy as jnp
from jax import lax
from jax.experimental import pallas as pl
from jax.experimental.pallas import tpu as pltpu
from jax.experimental.pallas import tpu_sc as plsc

N = 10000
E = 320000
D = 128
ED = 16
L = 3
G = 64

NT = 32
EPT = E // NT
CK = 125
CH = EPT // CK
CG = 8
NPAD = 10240
RPS = NPAD // 16

NB = 2000
NGRID = N // NB
BE = 2000


def _edge_emb_body(ea_ref, w_ref, b_ref, o0_ref, o1_ref, o2_ref):
    ea = ea_ref[...]
    for i, o_ref in enumerate((o0_ref, o1_ref, o2_ref)):
        o_ref[...] = (
            jnp.dot(ea, w_ref[i], preferred_element_type=jnp.float32)
            + b_ref[i][None, :]
        )


def _edge_emb(edge_attr, agg_eW, agg_eb):
    return pl.pallas_call(
        _edge_emb_body,
        grid=(E // BE,),
        in_specs=[
            pl.BlockSpec((BE, ED), lambda e: (e, 0)),
            pl.BlockSpec((L, ED, D), lambda e: (0, 0, 0)),
            pl.BlockSpec((L, D), lambda e: (0, 0)),
        ],
        out_specs=[pl.BlockSpec((BE, D), lambda e: (e, 0))] * L,
        out_shape=[jax.ShapeDtypeStruct((E, D), jnp.float32)] * L,
    )(edge_attr, agg_eW, agg_eb)


def _prep_body(x_ref, vn_ref, o_ref):
    o_ref[...] = x_ref[...] + vn_ref[0][None, :]


def _prep(x, vn_emb):
    return pl.pallas_call(
        _prep_body,
        grid=(NGRID,),
        in_specs=[
            pl.BlockSpec((NB, D), lambda n: (n, 0)),
            pl.BlockSpec((L, D), lambda n: (0, 0)),
        ],
        out_specs=pl.BlockSpec((NB, D), lambda n: (n, 0)),
        out_shape=jax.ShapeDtypeStruct((N, D), jnp.float32),
    )(x, vn_emb)


_SC_MESH = plsc.VectorSubcoreMesh(core_axis_name="c", subcore_axis_name="s")


@functools.partial(
    pl.kernel,
    out_type=jax.ShapeDtypeStruct((2, NPAD, D), jnp.float32),
    mesh=_SC_MESH,
    compiler_params=pltpu.CompilerParams(use_tc_tiling_on_sc=False),
    scratch_types=[
        pltpu.VMEM((CG, CK), jnp.int32),
        pltpu.VMEM((CG, CK), jnp.int32),
        pltpu.VMEM((CK, D), jnp.float32),
        pltpu.VMEM((CK, D), jnp.float32),
        pltpu.VMEM_SHARED((NPAD, D), jnp.float32),
        pltpu.SemaphoreType.DMA,
    ],
)
def _sc_aggregate(h_hbm, eemb_hbm, src_hbm, dst_hbm, zero_hbm, out_hbm,
                  src_v, dst_v, rows_v, emb_v, acc, sem):
    c = lax.axis_index("c")
    s = lax.axis_index("s")
    tile = c * 16 + s
    pltpu.sync_copy(zero_hbm.at[pl.ds(s * RPS, RPS)], acc.at[pl.ds(s * RPS, RPS)])
    plsc.subcore_barrier()

    ebase = tile * EPT

    def group_body(g, carry):
        pltpu.sync_copy(src_hbm.at[tile, pl.ds(g * CG, CG)], src_v)
        pltpu.sync_copy(dst_hbm.at[tile, pl.ds(g * CG, CG)], dst_v)

        def chunk_body(j, carry1):
            pltpu.async_copy(h_hbm.at[src_v.at[j]], rows_v, sem).wait()
            pltpu.sync_copy(
                eemb_hbm.at[pl.ds(ebase + (g * CG + j) * CK, CK)], emb_v)

            def row_body(r, carry2):
                for cc in range(D // 16):
                    sl = pl.ds(cc * 16, 16)
                    rows_v[r, sl] = jnp.maximum(rows_v[r, sl] + emb_v[r, sl],
                                                0.0)
                return carry2

            lax.fori_loop(0, CK, row_body, 0, unroll=False)
            pltpu.sync_copy(rows_v, acc.at[dst_v.at[j]], add=True)
            return carry1

        lax.fori_loop(0, CG, chunk_body, 0, unroll=False)
        return carry

    lax.fori_loop(0, CH // CG, group_body, 0, unroll=False)
    plsc.subcore_barrier()
    pltpu.sync_copy(acc.at[pl.ds(s * RPS, RPS)],
                    out_hbm.at[c, pl.ds(s * RPS, RPS)])


def _layer_body(h_ref, agg_ref, w1_ref, b1_ref, w2_ref, b2_ref, vn_ref,
                bt_ref, hout_ref, g_ref):
    z_in = h_ref[...] + agg_ref[0] + agg_ref[1]
    z = jnp.maximum(
        jnp.dot(z_in, w1_ref[...], preferred_element_type=jnp.float32)
        + b1_ref[...], 0.0)
    h_new = (jnp.dot(z, w2_ref[...], preferred_element_type=jnp.float32)
             + b2_ref[...])
    hout_ref[...] = h_new + vn_ref[...]
    onehot = (bt_ref[0, 0, :][None, :]
              == lax.broadcasted_iota(jnp.int32, (G, NB), 0)
              ).astype(jnp.float32)

    @pl.when(pl.program_id(0) == 0)
    def _():
        g_ref[...] = jnp.zeros_like(g_ref)

    g_ref[...] += jnp.dot(onehot, h_new, preferred_element_type=jnp.float32)


def _layer(h_cur, agg2, w1, b1, w2, b2, vn_next, batch2):
    return pl.pallas_call(
        _layer_body,
        grid=(NGRID,),
        in_specs=[
            pl.BlockSpec((NB, D), lambda n: (n, 0)),
            pl.BlockSpec((2, NB, D), lambda n: (0, n, 0)),

            pl.BlockSpec((D, D), lambda n: (0, 0)),
            pl.BlockSpec((1, D), lambda n: (0, 0)),
            pl.BlockSpec((D, D), lambda n: (0, 0)),
            pl.BlockSpec((1, D), lambda n: (0, 0)),
            pl.BlockSpec((1, D), lambda n: (0, 0)),
            pl.BlockSpec((1, 1, NB), lambda n: (n, 0, 0)),
        ],
        out_specs=[
            pl.BlockSpec((NB, D), lambda n: (n, 0)),
            pl.BlockSpec((G, D), lambda n: (0, 0)),
        ],
        out_shape=[
            jax.ShapeDtypeStruct((N, D), jnp.float32),
            jax.ShapeDtypeStruct((G, D), jnp.float32),
        ],
    )(h_cur, agg2, w1, b1, w2, b2, vn_next, batch2)


def _layer_norm(h, g, b):
    mu = jnp.mean(h, axis=-1, keepdims=True)
    var = jnp.mean((h - mu) ** 2, axis=-1, keepdims=True)
    return (h - mu) * lax.rsqrt(var + 1e-5) * g + b


def _final_body(g0_ref, g1_ref, g2_ref, w0_ref, b0_ref, ln_g0_ref, ln_b0_ref,
                w1_ref, b1_ref, ln_g1_ref, ln_b1_ref, we_ref, be_ref,
                wo_ref, bo_ref, out_ref):
    q = (jnp.dot(g0_ref[...], w0_ref[pl.ds(0, D), :],
                 preferred_element_type=jnp.float32)
         + jnp.dot(g1_ref[...], w0_ref[pl.ds(D, D), :],
                   preferred_element_type=jnp.float32)
         + jnp.dot(g2_ref[...], w0_ref[pl.ds(2 * D, D), :],
                   preferred_element_type=jnp.float32)
         + b0_ref[...])
    q = jnp.maximum(_layer_norm(q, ln_g0_ref[...], ln_b0_ref[...]), 0.0)
    q = jnp.dot(q, w1_ref[...], preferred_element_type=jnp.float32) + b1_ref[...]
    q = jnp.maximum(_layer_norm(q, ln_g1_ref[...], ln_b1_ref[...]), 0.0)
    emb = jnp.dot(q, we_ref[...], preferred_element_type=jnp.float32) + be_ref[...]
    out_ref[...] = (jnp.dot(emb, wo_ref[...], preferred_element_type=jnp.float32)
                    + bo_ref[...])


def _final(g0, g1, g2, w0, b0, ln_g0, ln_b0, w1, b1, ln_g1, ln_b1,
           we, be, wo, bo):
    return pl.pallas_call(
        _final_body,
        out_shape=jax.ShapeDtypeStruct((G, 1), jnp.float32),
    )(g0, g1, g2, w0, b0, ln_g0, ln_b0, w1, b1, ln_g1, ln_b1, we, be, wo, bo)


def kernel(x, edge_index, edge_attr, batch, vn_emb, vn_W1, vn_b1, vn_g, vn_bt,
           vn_W2, vn_b2, agg_eW, agg_eb, agg_W1, agg_b1, agg_W2, agg_b2,
           lin_W0, lin_b0, lin_g0, lin_bt0, lin_W1, lin_b1, lin_g1, lin_bt1,
           emb_W, emb_b, out_W, out_b):
    src_r = edge_index[0].reshape(NT, CH, CK).astype(jnp.int32)
    dst_r = edge_index[1].reshape(NT, CH, CK).astype(jnp.int32)
    batch2 = batch.reshape(NGRID, 1, NB).astype(jnp.int32)
    zero_nd = jnp.zeros((NPAD, D), jnp.float32)

    eembs = _edge_emb(edge_attr, agg_eW, agg_eb)
    h_cur = _prep(x, vn_emb)

    gs = []
    for i in range(L):
        agg2 = _sc_aggregate(h_cur, eembs[i], src_r, dst_r, zero_nd)
        if i + 1 < L:
            vn_next = vn_emb[i + 1].reshape(1, D)
        else:
            vn_next = jnp.zeros((1, D), jnp.float32)
        h_cur, g_i = _layer(h_cur, agg2, agg_W1[i], agg_b1[i].reshape(1, D),
                            agg_W2[i], agg_b2[i].reshape(1, D), vn_next,
                            batch2)
        gs.append(g_i)

    return _final(gs[0], gs[1], gs[2], lin_W0, lin_b0.reshape(1, D),
                  lin_g0.reshape(1, D), lin_bt0.reshape(1, D), lin_W1,
                  lin_b1.reshape(1, D), lin_g1.reshape(1, D),
                  lin_bt1.reshape(1, D), emb_W, emb_b.reshape(1, D),
                  out_W, out_b.reshape(1, 1))
